# baseline (speedup 1.0000x reference)
import jax
import jax.numpy as jnp
import numpy as np
from jax import lax
from jax.experimental import pallas as pl
from jax.experimental.pallas import tpu as pltpu
from jax.experimental.pallas import tpu_sc as plsc

_N, _Q, _C = 2, 5440, 256
_L, _H, _P = 4, 8, 4
_D = _C // _H
_SPATIAL = ((64, 64), (32, 32), (16, 16), (8, 8))
_HW = sum(h * w for h, w in _SPATIAL)
_QH = _N * _Q * _H
_QB = 320
_NQB = _Q // _QB
_MB = 640

_WV = np.repeat(np.array([w for (h, w) in _SPATIAL], np.float32), _P)
_HV = np.repeat(np.array([h for (h, w) in _SPATIAL], np.float32), _P)
_STARTV = np.repeat(
    np.cumsum([0] + [h * w for h, w in _SPATIAL])[:-1].astype(np.float32), _P
)
_FCONST = np.zeros((8, 16), np.float32)
_FCONST[0] = _WV
_FCONST[1] = _HV
_FCONST[2] = 1.0 / _WV
_FCONST[3] = 1.0 / _HV
_FCONST[4] = _STARTV

_OFF_PERM = np.empty(_C, np.int64)
for _h in range(_H):
    for _xy in range(2):
        for _l in range(_L):
            for _p in range(_P):
                _OFF_PERM[_h * 32 + _xy * 16 + _l * 4 + _p] = (
                    ((_h * _L + _l) * _P + _p) * 2 + _xy
                )

_NW = 32
_PW = _QH // _NW
_CH = 16
_NCHUNK = _PW // _CH


def _mm_body(x_ref, w_ref, b_ref, o_ref):
    o_ref[...] = (
        jnp.dot(x_ref[...], w_ref[...], preferred_element_type=jnp.float32)
        + b_ref[...]
    )


def _mm(x, w, b):
    m, k = x.shape
    n = w.shape[1]
    return pl.pallas_call(
        _mm_body,
        grid=(m // _MB,),
        in_specs=[
            pl.BlockSpec((_MB, k), lambda i: (i, 0)),
            pl.BlockSpec((k, n), lambda i: (0, 0)),
            pl.BlockSpec((1, n), lambda i: (0, 0)),
        ],
        out_specs=pl.BlockSpec((_MB, n), lambda i: (i, 0)),
        out_shape=jax.ShapeDtypeStruct((m, n), jnp.float32),
    )(x, w, b.reshape(1, n))


def _prep_body(q_ref, rx_ref, ry_ref, woff_ref, boff_ref, wattn_ref, battn_ref,
               fc_ref, idx_ref, wgt_ref):
    g = pl.program_id(0)
    n = g // _NQB
    q = q_ref[0]
    off = (
        jnp.dot(q, woff_ref[...], preferred_element_type=jnp.float32)
        + boff_ref[...]
    )
    att = (
        jnp.dot(q, wattn_ref[...], preferred_element_type=jnp.float32)
        + battn_ref[...]
    )
    rx = rx_ref[0]
    ry = ry_ref[0]
    fc = fc_ref[...]
    wv = fc[0:1, :]
    hv = fc[1:2, :]
    winv = fc[2:3, :]
    hinv = fc[3:4, :]
    wvi = wv.astype(jnp.int32)
    hvi = hv.astype(jnp.int32)
    startv = fc[4:5, :].astype(jnp.int32)
    for h in range(_H):
        oh = off[:, h * 32:(h + 1) * 32]
        ox = oh[:, 0:16]
        oy = oh[:, 16:32]
        a = att[:, h * 16:(h + 1) * 16]
        m = jnp.maximum(a[:, 0:8], a[:, 8:16])
        m = jnp.maximum(m[:, 0:4], m[:, 4:8])
        m = jnp.maximum(m[:, 0:2], m[:, 2:4])
        m = jnp.maximum(m[:, 0:1], m[:, 1:2])
        e = jnp.exp(a - m)
        s = e[:, 0:8] + e[:, 8:16]
        s = s[:, 0:4] + s[:, 4:8]
        s = s[:, 0:2] + s[:, 2:4]
        s = s[:, 0:1] + s[:, 1:2]
        sm = e / s
        x = (rx + ox * winv) * wv - 0.5
        y = (ry + oy * hinv) * hv - 0.5
        x0f = jnp.floor(x)
        y0f = jnp.floor(y)
        lx = x - x0f
        ly = y - y0f
        x0 = x0f.astype(jnp.int32)
        y0 = y0f.astype(jnp.int32)
        xbase = jnp.clip(x0, 0, wvi - 1)
        wx = []
        for e in (0, 1):
            x_e = xbase + e
            wx.append(
                jnp.where(x_e == x0, 1.0 - lx, jnp.where(x_e == x0 + 1, lx, 0.0))
                * (x_e < wvi).astype(jnp.float32)
            )
        idx_parts = []
        w_parts = []
        for cy in (0, 1):
            y_c = y0 + cy
            wy = (ly if cy else (1.0 - ly)) * (
                (y_c >= 0) & (y_c < hvi)
            ).astype(jnp.float32)
            yc = jnp.clip(y_c, 0, hvi - 1)
            sp = yc * wvi + xbase + startv + n * _HW
            idx_parts.append(sp * _H + h)
            w_parts.append(sm * wy * wx[0])
            w_parts.append(sm * wy * wx[1])
        idx_ref[0, :, h * 32:(h + 1) * 32] = jnp.concatenate(idx_parts, axis=1)
        wgt_ref[0, :, h * 64:(h + 1) * 64] = jnp.concatenate(
            [w_parts[0], w_parts[1], w_parts[2], w_parts[3]], axis=1
        )


def _prep(query3, rx3, ry3, woff, boff, wattn, battn):
    g = _N * _NQB
    return pl.pallas_call(
        _prep_body,
        grid=(g,),
        in_specs=[
            pl.BlockSpec((1, _QB, _C), lambda i: (i, 0, 0)),
            pl.BlockSpec((1, _QB, 16), lambda i: (i, 0, 0)),
            pl.BlockSpec((1, _QB, 16), lambda i: (i, 0, 0)),
            pl.BlockSpec((_C, _C), lambda i: (0, 0)),
            pl.BlockSpec((1, _C), lambda i: (0, 0)),
            pl.BlockSpec((_C, 128), lambda i: (0, 0)),
            pl.BlockSpec((1, 128), lambda i: (0, 0)),
            pl.BlockSpec((8, 16), lambda i: (0, 0)),
        ],
        out_specs=[
            pl.BlockSpec((1, _QB, 256), lambda i: (i, 0, 0)),
            pl.BlockSpec((1, _QB, 512), lambda i: (i, 0, 0)),
        ],
        out_shape=[
            jax.ShapeDtypeStruct((g, _QB, 256), jnp.int32),
            jax.ShapeDtypeStruct((g, _QB, 512), jnp.float32),
        ],
    )(query3, rx3, ry3, woff, boff, wattn, battn, jnp.asarray(_FCONST))


def _sc_body(idx_hbm, wgt_hbm, val_hbm, out_hbm,
             idx_a, idx_b, wgt_a, wgt_b, rows_a, rows_b, out_v,
             sem_a_io, sem_b_io, sem_a_g, sem_b_g):
    cid = lax.axis_index("c")
    sid = lax.axis_index("s")
    wid = sid * 2 + cid
    base0 = wid * _PW
    iota = lax.iota(jnp.int32, 16)
    zero = jnp.zeros((16,), jnp.float32)

    def chunk_off(c):
        c = lax.rem(c, _NCHUNK)
        return pl.multiple_of(base0 + c * _CH, _CH)

    def fire_io(c, idx_v, wgt_v, sem):
        ib = chunk_off(c)
        row128 = pl.multiple_of((ib * 32) // 128, 4)
        cp1 = pltpu.make_async_copy(idx_hbm.at[pl.ds(row128, 4)], idx_v, sem)
        cp2 = pltpu.make_async_copy(
            wgt_hbm.at[pl.ds(pl.multiple_of(ib * 64, 1024), _CH * 64)],
            wgt_v.at[pl.ds(0, _CH * 64)], sem,
        )
        cp1.start()
        cp2.start()

    def wait_io(c, idx_v, wgt_v, sem):
        ib = chunk_off(c)
        row128 = pl.multiple_of((ib * 32) // 128, 4)
        pltpu.make_async_copy(idx_hbm.at[pl.ds(row128, 4)], idx_v, sem).wait()
        pltpu.make_async_copy(
            wgt_hbm.at[pl.ds(pl.multiple_of(ib * 64, 1024), _CH * 64)],
            wgt_v.at[pl.ds(0, _CH * 64)], sem,
        ).wait()

    def fire_gathers(idx_v, rows_v, sem):
        for j in range(4):
            pltpu.make_async_copy(
                val_hbm.at[idx_v.at[j]], rows_v.at[pl.ds(j * 128, 128)], sem
            ).start()

    def wait_gathers(idx_v, rows_v, sem):
        for j in range(4):
            pltpu.make_async_copy(
                val_hbm.at[idx_v.at[j]], rows_v.at[pl.ds(j * 128, 128)], sem
            ).wait()

    def compute(c, wgt_v, rows_v):
        ib = chunk_off(c)
        for i0 in (0, 4, 8, 12):

            def jbody(jj, accs, i0=i0):
                jbase = lax.shift_left(lax.bitwise_and(jj, 16), 1) + \
                    lax.bitwise_and(jj, 15)
                new = []
                for k in range(4):
                    i = i0 + k
                    fl = i * 32 + jj
                    w0 = wgt_v[pl.ds(i * 64 + jbase, 16)][0]
                    w1 = wgt_v[pl.ds(i * 64 + 16 + jbase, 16)][0]
                    lo = accs[2 * k] + w0 * rows_v[fl, 0:16] \
                        + w1 * rows_v[fl, 32:48]
                    hi = accs[2 * k + 1] + w0 * rows_v[fl, 16:32] \
                        + w1 * rows_v[fl, 48:64]
                    new += [lo, hi]
                return tuple(new)

            accs = lax.fori_loop(0, 32, jbody, (zero,) * 8, unroll=2)
            for k in range(4):
                out_v[pl.ds((i0 + k) * 32, 16)] = accs[2 * k]
                out_v[pl.ds((i0 + k) * 32 + 16, 16)] = accs[2 * k + 1]
        pltpu.sync_copy(
            out_v, out_hbm.at[pl.ds(pl.multiple_of(ib * 32, 512), _CH * 32)]
        )

    fire_io(0, idx_a, wgt_a, sem_a_io)
    fire_io(1, idx_b, wgt_b, sem_b_io)
    wait_io(0, idx_a, wgt_a, sem_a_io)
    fire_gathers(idx_a, rows_a, sem_a_g)

    def step(k, carry):
        t = k * 2
        wait_io(t + 1, idx_b, wgt_b, sem_b_io)
        fire_gathers(idx_b, rows_b, sem_b_g)
        wait_gathers(idx_a, rows_a, sem_a_g)
        compute(t, wgt_a, rows_a)
        fire_io(t + 2, idx_a, wgt_a, sem_a_io)

        wait_io(t + 2, idx_a, wgt_a, sem_a_io)
        fire_gathers(idx_a, rows_a, sem_a_g)
        wait_gathers(idx_b, rows_b, sem_b_g)
        compute(t + 1, wgt_b, rows_b)
        fire_io(t + 3, idx_b, wgt_b, sem_b_io)
        return carry

    lax.fori_loop(0, _NCHUNK // 2, step, 0)
    wait_gathers(idx_a, rows_a, sem_a_g)
    wait_io(_NCHUNK + 1, idx_b, wgt_b, sem_b_io)


def _sc_gather(idx2, wgt2, val_rows):
    mesh = plsc.VectorSubcoreMesh(core_axis_name="c", subcore_axis_name="s")
    return pl.kernel(
        _sc_body,
        out_type=jax.ShapeDtypeStruct((_QH * _D,), jnp.float32),
        mesh=mesh,
        compiler_params=pltpu.CompilerParams(
            needs_layout_passes=False, use_tc_tiling_on_sc=False
        ),
        scratch_types=[
            pltpu.VMEM((4, 128), jnp.int32),
            pltpu.VMEM((4, 128), jnp.int32),
            pltpu.VMEM((_CH * 64 + 32,), jnp.float32),
            pltpu.VMEM((_CH * 64 + 32,), jnp.float32),
            pltpu.VMEM((_CH * 32, 2 * _D), jnp.float32),
            pltpu.VMEM((_CH * 32, 2 * _D), jnp.float32),
            pltpu.VMEM((_CH * _D,), jnp.float32),
            pltpu.SemaphoreType.DMA,
            pltpu.SemaphoreType.DMA,
            pltpu.SemaphoreType.DMA,
            pltpu.SemaphoreType.DMA,
        ],
    )(idx2, wgt2, val_rows)


def kernel(query, reference_points, input_flatten, input_spatial_shapes,
           input_level_start_index, W_value, b_value, W_off, b_off, W_attn,
           b_attn, W_out, b_out):
    val = _mm(input_flatten.reshape(_N * _HW, _C), W_value, b_value)
    val_next = jnp.concatenate([val[1:], val[-1:]], axis=0)
    val_rows = jnp.concatenate(
        [val.reshape(_N * _HW, _H, _D), val_next.reshape(_N * _HW, _H, _D)],
        axis=2,
    ).reshape(_QH, 2 * _D)

    rx3 = jnp.repeat(reference_points[..., 0], _P, axis=2).reshape(
        _N * _NQB, _QB, 16
    )
    ry3 = jnp.repeat(reference_points[..., 1], _P, axis=2).reshape(
        _N * _NQB, _QB, 16
    )
    query3 = query.reshape(_N * _NQB, _QB, _C)
    woff = W_off[:, _OFF_PERM]
    boff = b_off[_OFF_PERM].reshape(1, _C)

    idx_out, wgt_out = _prep(
        query3, rx3, ry3, woff, boff, W_attn, b_attn.reshape(1, 128)
    )
    idx2 = idx_out.reshape(_QH * 32 // 128, 128)
    wgt2 = wgt_out.reshape(_QH * 64)

    sampled = _sc_gather(idx2, wgt2, val_rows)

    out = _mm(sampled.reshape(_N * _Q, _C), W_out, b_out)
    return out.reshape(_N, _Q, _C)


# speedup vs baseline: 1.2125x; 1.2125x over previous
import jax
import jax.numpy as jnp
import numpy as np
from jax import lax
from jax.experimental import pallas as pl
from jax.experimental.pallas import tpu as pltpu
from jax.experimental.pallas import tpu_sc as plsc

_N, _Q, _C = 2, 5440, 256
_L, _H, _P = 4, 8, 4
_D = _C // _H
_SPATIAL = ((64, 64), (32, 32), (16, 16), (8, 8))
_HW = sum(h * w for h, w in _SPATIAL)
_QH = _N * _Q * _H
_QB = 320
_NQB = _Q // _QB
_MB = 640

_WV = np.repeat(np.array([w for (h, w) in _SPATIAL], np.float32), _P)
_HV = np.repeat(np.array([h for (h, w) in _SPATIAL], np.float32), _P)
_STARTV = np.repeat(
    np.cumsum([0] + [h * w for h, w in _SPATIAL])[:-1].astype(np.float32), _P
)
_FCONST = np.zeros((8, 16), np.float32)
_FCONST[0] = _WV
_FCONST[1] = _HV
_FCONST[2] = 1.0 / _WV
_FCONST[3] = 1.0 / _HV
_FCONST[4] = _STARTV

_OFF_PERM = np.empty(_C, np.int64)
for _h in range(_H):
    for _xy in range(2):
        for _l in range(_L):
            for _p in range(_P):
                _OFF_PERM[_h * 32 + _xy * 16 + _l * 4 + _p] = (
                    ((_h * _L + _l) * _P + _p) * 2 + _xy
                )

_VAL_PERM = np.empty(_C, np.int64)
for _h in range(_H):
    for _t in range(16):
        _VAL_PERM[_h * 32 + 2 * _t] = _h * 32 + _t
        _VAL_PERM[_h * 32 + 2 * _t + 1] = _h * 32 + 16 + _t

_NW = 32
_PW = _QH // _NW
_CH = 16
_NCHUNK = _PW // _CH


def _mm_body(x_ref, w_ref, b_ref, o_ref):
    o_ref[...] = (
        jnp.dot(x_ref[...], w_ref[...], preferred_element_type=jnp.float32)
        + b_ref[...]
    )


def _mm(x, w, b):
    m, k = x.shape
    n = w.shape[1]
    return pl.pallas_call(
        _mm_body,
        grid=(m // _MB,),
        in_specs=[
            pl.BlockSpec((_MB, k), lambda i: (i, 0)),
            pl.BlockSpec((k, n), lambda i: (0, 0)),
            pl.BlockSpec((1, n), lambda i: (0, 0)),
        ],
        out_specs=pl.BlockSpec((_MB, n), lambda i: (i, 0)),
        out_shape=jax.ShapeDtypeStruct((m, n), jnp.float32),
    )(x, w, b.reshape(1, n))


def _prep_body(q_ref, rx_ref, ry_ref, woff_ref, boff_ref, wattn_ref, battn_ref,
               fc_ref, idx_ref, wgt_ref):
    g = pl.program_id(0)
    n = g // _NQB
    q = q_ref[0]
    off = (
        jnp.dot(q, woff_ref[...], preferred_element_type=jnp.float32)
        + boff_ref[...]
    )
    att = (
        jnp.dot(q, wattn_ref[...], preferred_element_type=jnp.float32)
        + battn_ref[...]
    )
    rx = rx_ref[0]
    ry = ry_ref[0]
    fc = fc_ref[...]
    wv = fc[0:1, :]
    hv = fc[1:2, :]
    winv = fc[2:3, :]
    hinv = fc[3:4, :]
    wvi = wv.astype(jnp.int32)
    hvi = hv.astype(jnp.int32)
    startv = fc[4:5, :].astype(jnp.int32)
    for h in range(_H):
        oh = off[:, h * 32:(h + 1) * 32]
        ox = oh[:, 0:16]
        oy = oh[:, 16:32]
        a = att[:, h * 16:(h + 1) * 16]
        m = jnp.maximum(a[:, 0:8], a[:, 8:16])
        m = jnp.maximum(m[:, 0:4], m[:, 4:8])
        m = jnp.maximum(m[:, 0:2], m[:, 2:4])
        m = jnp.maximum(m[:, 0:1], m[:, 1:2])
        e = jnp.exp(a - m)
        s = e[:, 0:8] + e[:, 8:16]
        s = s[:, 0:4] + s[:, 4:8]
        s = s[:, 0:2] + s[:, 2:4]
        s = s[:, 0:1] + s[:, 1:2]
        sm = e / s
        x = (rx + ox * winv) * wv - 0.5
        y = (ry + oy * hinv) * hv - 0.5
        x0f = jnp.floor(x)
        y0f = jnp.floor(y)
        lx = x - x0f
        ly = y - y0f
        x0 = x0f.astype(jnp.int32)
        y0 = y0f.astype(jnp.int32)
        xbase = jnp.clip(x0, 0, wvi - 1)
        wx = []
        for e in (0, 1):
            x_e = xbase + e
            wx.append(
                jnp.where(x_e == x0, 1.0 - lx, jnp.where(x_e == x0 + 1, lx, 0.0))
                * (x_e < wvi).astype(jnp.float32)
            )
        idx_parts = []
        w_parts = []
        for cy in (0, 1):
            y_c = y0 + cy
            wy = (ly if cy else (1.0 - ly)) * (
                (y_c >= 0) & (y_c < hvi)
            ).astype(jnp.float32)
            yc = jnp.clip(y_c, 0, hvi - 1)
            sp = yc * wvi + xbase + startv + n * _HW
            idx_parts.append(sp * _H + h)
            w_parts.append(sm * wy * wx[0])
            w_parts.append(sm * wy * wx[1])
        idx_ref[0, :, h * 32:(h + 1) * 32] = jnp.concatenate(idx_parts, axis=1)
        wgt_ref[0, :, h * 64:(h + 1) * 64] = jnp.concatenate(
            [w_parts[0], w_parts[1], w_parts[2], w_parts[3]], axis=1
        )


def _prep(query3, rx3, ry3, woff, boff, wattn, battn):
    g = _N * _NQB
    return pl.pallas_call(
        _prep_body,
        grid=(g,),
        in_specs=[
            pl.BlockSpec((1, _QB, _C), lambda i: (i, 0, 0)),
            pl.BlockSpec((1, _QB, 16), lambda i: (i, 0, 0)),
            pl.BlockSpec((1, _QB, 16), lambda i: (i, 0, 0)),
            pl.BlockSpec((_C, _C), lambda i: (0, 0)),
            pl.BlockSpec((1, _C), lambda i: (0, 0)),
            pl.BlockSpec((_C, 128), lambda i: (0, 0)),
            pl.BlockSpec((1, 128), lambda i: (0, 0)),
            pl.BlockSpec((8, 16), lambda i: (0, 0)),
        ],
        out_specs=[
            pl.BlockSpec((1, _QB, 256), lambda i: (i, 0, 0)),
            pl.BlockSpec((1, _QB, 512), lambda i: (i, 0, 0)),
        ],
        out_shape=[
            jax.ShapeDtypeStruct((g, _QB, 256), jnp.int32),
            jax.ShapeDtypeStruct((g, _QB, 512), jnp.float32),
        ],
    )(query3, rx3, ry3, woff, boff, wattn, battn, jnp.asarray(_FCONST))


def _sc_body(idx_hbm, wgt_hbm, val_hbm, out_hbm,
             idx_a, idx_b, wgt_a, wgt_b, rows_a, rows_b, out_v,
             sem_a_io, sem_b_io, sem_a_g, sem_b_g):
    cid = lax.axis_index("c")
    sid = lax.axis_index("s")
    wid = sid * 2 + cid
    base0 = wid * _PW
    iota = lax.iota(jnp.int32, 16)
    zero = jnp.zeros((16,), jnp.float32)

    def chunk_off(c):
        c = lax.rem(c, _NCHUNK)
        return pl.multiple_of(base0 + c * _CH, _CH)

    def fire_io(c, idx_v, wgt_v, sem):
        ib = chunk_off(c)
        row128 = pl.multiple_of((ib * 32) // 128, 4)
        cp1 = pltpu.make_async_copy(idx_hbm.at[pl.ds(row128, 4)], idx_v, sem)
        cp2 = pltpu.make_async_copy(
            wgt_hbm.at[pl.ds(pl.multiple_of(ib * 64, 1024), _CH * 64)],
            wgt_v.at[pl.ds(0, _CH * 64)], sem,
        )
        cp1.start()
        cp2.start()

    def wait_io(c, idx_v, wgt_v, sem):
        ib = chunk_off(c)
        row128 = pl.multiple_of((ib * 32) // 128, 4)
        pltpu.make_async_copy(idx_hbm.at[pl.ds(row128, 4)], idx_v, sem).wait()
        pltpu.make_async_copy(
            wgt_hbm.at[pl.ds(pl.multiple_of(ib * 64, 1024), _CH * 64)],
            wgt_v.at[pl.ds(0, _CH * 64)], sem,
        ).wait()

    def fire_gathers(idx_v, rows_v, sem):
        for j in range(4):
            pltpu.make_async_copy(
                val_hbm.at[idx_v.at[j]], rows_v.at[pl.ds(j * 128, 128)], sem
            ).start()

    def wait_gathers(idx_v, rows_v, sem):
        for j in range(4):
            pltpu.make_async_copy(
                val_hbm.at[idx_v.at[j]], rows_v.at[pl.ds(j * 128, 128)], sem
            ).wait()

    def compute(c, wgt_v, rows_v):
        ib = chunk_off(c)
        for i0 in (0, 4, 8, 12):

            def jbody(jj, accs, i0=i0):
                jbase = lax.shift_left(lax.bitwise_and(jj, 16), 1) + \
                    lax.bitwise_and(jj, 15)
                new = []
                for k in range(4):
                    i = i0 + k
                    fl = i * 32 + jj
                    w0 = wgt_v[pl.ds(i * 64 + jbase, 16)][0]
                    w1 = wgt_v[pl.ds(i * 64 + 16 + jbase, 16)][0]
                    l0, l1 = plsc.unpack(
                        rows_v[fl, 0:32], format=plsc.PackFormat.INTERLEAVED
                    )
                    r0, r1 = plsc.unpack(
                        rows_v[fl, 32:64], format=plsc.PackFormat.INTERLEAVED
                    )
                    lo = accs[2 * k] + w0 * l0 + w1 * r0
                    hi = accs[2 * k + 1] + w0 * l1 + w1 * r1
                    new += [lo, hi]
                return tuple(new)

            accs = lax.fori_loop(0, 32, jbody, (zero,) * 8, unroll=2)
            for k in range(4):
                out_v[pl.ds((i0 + k) * 32, 16)] = accs[2 * k]
                out_v[pl.ds((i0 + k) * 32 + 16, 16)] = accs[2 * k + 1]
        pltpu.sync_copy(
            out_v, out_hbm.at[pl.ds(pl.multiple_of(ib * 32, 512), _CH * 32)]
        )

    fire_io(0, idx_a, wgt_a, sem_a_io)
    fire_io(1, idx_b, wgt_b, sem_b_io)
    wait_io(0, idx_a, wgt_a, sem_a_io)
    fire_gathers(idx_a, rows_a, sem_a_g)

    def step(k, carry):
        t = k * 2
        wait_io(t + 1, idx_b, wgt_b, sem_b_io)
        fire_gathers(idx_b, rows_b, sem_b_g)
        wait_gathers(idx_a, rows_a, sem_a_g)
        compute(t, wgt_a, rows_a)
        fire_io(t + 2, idx_a, wgt_a, sem_a_io)

        wait_io(t + 2, idx_a, wgt_a, sem_a_io)
        fire_gathers(idx_a, rows_a, sem_a_g)
        wait_gathers(idx_b, rows_b, sem_b_g)
        compute(t + 1, wgt_b, rows_b)
        fire_io(t + 3, idx_b, wgt_b, sem_b_io)
        return carry

    lax.fori_loop(0, _NCHUNK // 2, step, 0)
    wait_gathers(idx_a, rows_a, sem_a_g)
    wait_io(_NCHUNK + 1, idx_b, wgt_b, sem_b_io)


def _sc_gather(idx2, wgt2, val_rows):
    mesh = plsc.VectorSubcoreMesh(core_axis_name="c", subcore_axis_name="s")
    return pl.kernel(
        _sc_body,
        out_type=jax.ShapeDtypeStruct((_QH * _D,), jnp.float32),
        mesh=mesh,
        compiler_params=pltpu.CompilerParams(
            needs_layout_passes=False, use_tc_tiling_on_sc=False
        ),
        scratch_types=[
            pltpu.VMEM((4, 128), jnp.int32),
            pltpu.VMEM((4, 128), jnp.int32),
            pltpu.VMEM((_CH * 64 + 32,), jnp.float32),
            pltpu.VMEM((_CH * 64 + 32,), jnp.float32),
            pltpu.VMEM((_CH * 32, 2 * _D), jnp.bfloat16),
            pltpu.VMEM((_CH * 32, 2 * _D), jnp.bfloat16),
            pltpu.VMEM((_CH * _D,), jnp.float32),
            pltpu.SemaphoreType.DMA,
            pltpu.SemaphoreType.DMA,
            pltpu.SemaphoreType.DMA,
            pltpu.SemaphoreType.DMA,
        ],
    )(idx2, wgt2, val_rows)


def kernel(query, reference_points, input_flatten, input_spatial_shapes,
           input_level_start_index, W_value, b_value, W_off, b_off, W_attn,
           b_attn, W_out, b_out):
    val = _mm(
        input_flatten.reshape(_N * _HW, _C), W_value[:, _VAL_PERM],
        b_value[_VAL_PERM],
    ).astype(jnp.bfloat16)
    val_next = jnp.concatenate([val[1:], val[-1:]], axis=0)
    val_rows = jnp.concatenate(
        [val.reshape(_N * _HW, _H, _D), val_next.reshape(_N * _HW, _H, _D)],
        axis=2,
    ).reshape(_QH, 2 * _D)

    rx3 = jnp.repeat(reference_points[..., 0], _P, axis=2).reshape(
        _N * _NQB, _QB, 16
    )
    ry3 = jnp.repeat(reference_points[..., 1], _P, axis=2).reshape(
        _N * _NQB, _QB, 16
    )
    query3 = query.reshape(_N * _NQB, _QB, _C)
    woff = W_off[:, _OFF_PERM]
    boff = b_off[_OFF_PERM].reshape(1, _C)

    idx_out, wgt_out = _prep(
        query3, rx3, ry3, woff, boff, W_attn, b_attn.reshape(1, 128)
    )
    idx2 = idx_out.reshape(_QH * 32 // 128, 128)
    wgt2 = wgt_out.reshape(_QH * 64)

    sampled = _sc_gather(idx2, wgt2, val_rows)

    out = _mm(sampled.reshape(_N * _Q, _C), W_out, b_out)
    return out.reshape(_N, _Q, _C)


# speedup vs baseline: 1.2793x; 1.0551x over previous
import jax
import jax.numpy as jnp
import numpy as np
from jax import lax
from jax.experimental import pallas as pl
from jax.experimental.pallas import tpu as pltpu
from jax.experimental.pallas import tpu_sc as plsc

_N, _Q, _C = 2, 5440, 256
_L, _H, _P = 4, 8, 4
_D = _C // _H
_SPATIAL = ((64, 64), (32, 32), (16, 16), (8, 8))
_HW = sum(h * w for h, w in _SPATIAL)
_QH = _N * _Q * _H
_QB = 320
_NQB = _Q // _QB
_MB = 640

_WV = np.repeat(np.array([w for (h, w) in _SPATIAL], np.float32), _P)
_HV = np.repeat(np.array([h for (h, w) in _SPATIAL], np.float32), _P)
_STARTV = np.repeat(
    np.cumsum([0] + [h * w for h, w in _SPATIAL])[:-1].astype(np.float32), _P
)
_FCONST = np.zeros((8, 16), np.float32)
_FCONST[0] = _WV
_FCONST[1] = _HV
_FCONST[2] = 1.0 / _WV
_FCONST[3] = 1.0 / _HV
_FCONST[4] = _STARTV

_OFF_PERM = np.empty(_C, np.int64)
for _h in range(_H):
    for _xy in range(2):
        for _l in range(_L):
            for _p in range(_P):
                _OFF_PERM[_h * 32 + _xy * 16 + _l * 4 + _p] = (
                    ((_h * _L + _l) * _P + _p) * 2 + _xy
                )

_VAL_PERM = np.empty(_C, np.int64)
for _h in range(_H):
    for _t in range(16):
        _VAL_PERM[_h * 32 + 2 * _t] = _h * 32 + _t
        _VAL_PERM[_h * 32 + 2 * _t + 1] = _h * 32 + 16 + _t

_NW = 32
_PW = _QH // _NW
_CH = 16
_NCHUNK = _PW // _CH


def _mm_body(x_ref, w_ref, b_ref, o_ref):
    o_ref[...] = (
        jnp.dot(x_ref[...], w_ref[...], preferred_element_type=jnp.float32)
        + b_ref[...]
    ).astype(o_ref.dtype)


def _mm(x, w, b, out_dtype=jnp.float32):
    m, k = x.shape
    n = w.shape[1]
    return pl.pallas_call(
        _mm_body,
        grid=(m // _MB,),
        in_specs=[
            pl.BlockSpec((_MB, k), lambda i: (i, 0)),
            pl.BlockSpec((k, n), lambda i: (0, 0)),
            pl.BlockSpec((1, n), lambda i: (0, 0)),
        ],
        out_specs=pl.BlockSpec((_MB, n), lambda i: (i, 0)),
        out_shape=jax.ShapeDtypeStruct((m, n), out_dtype),
    )(x, w, b.reshape(1, n))


def _prep_body(q_ref, rx_ref, ry_ref, woff_ref, boff_ref, wattn_ref, battn_ref,
               fc_ref, idx_ref, wgt_ref):
    g = pl.program_id(0)
    n = g // _NQB
    q = q_ref[0]
    off = (
        jnp.dot(q, woff_ref[...], preferred_element_type=jnp.float32)
        + boff_ref[...]
    )
    att = (
        jnp.dot(q, wattn_ref[...], preferred_element_type=jnp.float32)
        + battn_ref[...]
    )
    rx = rx_ref[0]
    ry = ry_ref[0]
    fc = fc_ref[...]
    wv = fc[0:1, :]
    hv = fc[1:2, :]
    winv = fc[2:3, :]
    hinv = fc[3:4, :]
    wvi = wv.astype(jnp.int32)
    hvi = hv.astype(jnp.int32)
    startv = fc[4:5, :].astype(jnp.int32)
    for h in range(_H):
        oh = off[:, h * 32:(h + 1) * 32]
        ox = oh[:, 0:16]
        oy = oh[:, 16:32]
        a = att[:, h * 16:(h + 1) * 16]
        m = jnp.maximum(a[:, 0:8], a[:, 8:16])
        m = jnp.maximum(m[:, 0:4], m[:, 4:8])
        m = jnp.maximum(m[:, 0:2], m[:, 2:4])
        m = jnp.maximum(m[:, 0:1], m[:, 1:2])
        e = jnp.exp(a - m)
        s = e[:, 0:8] + e[:, 8:16]
        s = s[:, 0:4] + s[:, 4:8]
        s = s[:, 0:2] + s[:, 2:4]
        s = s[:, 0:1] + s[:, 1:2]
        sm = e / s
        x = (rx + ox * winv) * wv - 0.5
        y = (ry + oy * hinv) * hv - 0.5
        x0f = jnp.floor(x)
        y0f = jnp.floor(y)
        lx = x - x0f
        ly = y - y0f
        x0 = x0f.astype(jnp.int32)
        y0 = y0f.astype(jnp.int32)
        idx_parts = []
        w_parts = []
        for dy, dx in ((0, 0), (0, 1), (1, 0), (1, 1)):
            xx = x0 + dx
            yy = y0 + dy
            valid = (xx >= 0) & (xx < wvi) & (yy >= 0) & (yy < hvi)
            xc = jnp.clip(xx, 0, wvi - 1)
            yc = jnp.clip(yy, 0, hvi - 1)
            sp = yc * wvi + xc + startv + n * _HW
            idx_parts.append(sp * _H + h)
            bw = (ly if dy else (1.0 - ly)) * (lx if dx else (1.0 - lx))
            w_parts.append(sm * bw * valid.astype(jnp.float32))
        idx_ref[0, :, h * 64:(h + 1) * 64] = jnp.concatenate(idx_parts, axis=1)
        wgt_ref[0, :, h * 64:(h + 1) * 64] = jnp.concatenate(w_parts, axis=1)


def _prep(query3, rx3, ry3, woff, boff, wattn, battn):
    g = _N * _NQB
    return pl.pallas_call(
        _prep_body,
        grid=(g,),
        in_specs=[
            pl.BlockSpec((1, _QB, _C), lambda i: (i, 0, 0)),
            pl.BlockSpec((1, _QB, 16), lambda i: (i, 0, 0)),
            pl.BlockSpec((1, _QB, 16), lambda i: (i, 0, 0)),
            pl.BlockSpec((_C, _C), lambda i: (0, 0)),
            pl.BlockSpec((1, _C), lambda i: (0, 0)),
            pl.BlockSpec((_C, 128), lambda i: (0, 0)),
            pl.BlockSpec((1, 128), lambda i: (0, 0)),
            pl.BlockSpec((8, 16), lambda i: (0, 0)),
        ],
        out_specs=[
            pl.BlockSpec((1, _QB, 512), lambda i: (i, 0, 0)),
            pl.BlockSpec((1, _QB, 512), lambda i: (i, 0, 0)),
        ],
        out_shape=[
            jax.ShapeDtypeStruct((g, _QB, 512), jnp.int32),
            jax.ShapeDtypeStruct((g, _QB, 512), jnp.float32),
        ],
    )(query3, rx3, ry3, woff, boff, wattn, battn, jnp.asarray(_FCONST))


def _sc_body(idx_hbm, wgt_hbm, val_hbm, out_hbm,
             idx_a, idx_b, wgt_a, wgt_b, rows_a, rows_b, out_v,
             sem_a_io, sem_b_io, sem_a_g, sem_b_g):
    cid = lax.axis_index("c")
    sid = lax.axis_index("s")
    wid = sid * 2 + cid
    base0 = wid * _PW
    iota = lax.iota(jnp.int32, 16)
    zero = jnp.zeros((16,), jnp.float32)

    def chunk_off(c):
        c = lax.rem(c, _NCHUNK)
        return pl.multiple_of(base0 + c * _CH, _CH)

    def fire_io(c, idx_v, wgt_v, sem):
        ib = chunk_off(c)
        row128 = pl.multiple_of((ib * 64) // 128, 8)
        cp1 = pltpu.make_async_copy(idx_hbm.at[pl.ds(row128, 8)], idx_v, sem)
        cp2 = pltpu.make_async_copy(
            wgt_hbm.at[pl.ds(pl.multiple_of(ib * 64, 1024), _CH * 64)],
            wgt_v.at[pl.ds(0, _CH * 64)], sem,
        )
        cp1.start()
        cp2.start()

    def wait_io(c, idx_v, wgt_v, sem):
        ib = chunk_off(c)
        row128 = pl.multiple_of((ib * 64) // 128, 8)
        pltpu.make_async_copy(idx_hbm.at[pl.ds(row128, 8)], idx_v, sem).wait()
        pltpu.make_async_copy(
            wgt_hbm.at[pl.ds(pl.multiple_of(ib * 64, 1024), _CH * 64)],
            wgt_v.at[pl.ds(0, _CH * 64)], sem,
        ).wait()

    def fire_gathers(idx_v, rows_v, sem):
        for j in range(8):
            pltpu.make_async_copy(
                val_hbm.at[idx_v.at[j]], rows_v.at[pl.ds(j * 128, 128)], sem
            ).start()

    def wait_gathers(idx_v, rows_v, sem):
        for j in range(8):
            pltpu.make_async_copy(
                val_hbm.at[idx_v.at[j]], rows_v.at[pl.ds(j * 128, 128)], sem
            ).wait()

    def compute(c, wgt_v, rows_v):
        ib = chunk_off(c)
        for i0 in (0, 4, 8, 12):

            def jbody(j, accs, i0=i0):
                new = []
                for k in range(4):
                    i = i0 + k
                    fl = i * 64 + j
                    w0 = wgt_v[pl.ds(fl, 16)][0]
                    l0, l1 = plsc.unpack(
                        rows_v[fl, 0:32], format=plsc.PackFormat.INTERLEAVED
                    )
                    lo = accs[2 * k] + w0 * l0
                    hi = accs[2 * k + 1] + w0 * l1
                    new += [lo, hi]
                return tuple(new)

            accs = lax.fori_loop(0, 64, jbody, (zero,) * 8, unroll=2)
            for k in range(4):
                out_v[pl.ds((i0 + k) * 32, 16)] = accs[2 * k]
                out_v[pl.ds((i0 + k) * 32 + 16, 16)] = accs[2 * k + 1]
        pltpu.sync_copy(
            out_v, out_hbm.at[pl.ds(pl.multiple_of(ib * 32, 512), _CH * 32)]
        )

    fire_io(0, idx_a, wgt_a, sem_a_io)
    fire_io(1, idx_b, wgt_b, sem_b_io)
    wait_io(0, idx_a, wgt_a, sem_a_io)
    fire_gathers(idx_a, rows_a, sem_a_g)

    def step(k, carry):
        t = k * 2
        wait_io(t + 1, idx_b, wgt_b, sem_b_io)
        fire_gathers(idx_b, rows_b, sem_b_g)
        wait_gathers(idx_a, rows_a, sem_a_g)
        compute(t, wgt_a, rows_a)
        fire_io(t + 2, idx_a, wgt_a, sem_a_io)

        wait_io(t + 2, idx_a, wgt_a, sem_a_io)
        fire_gathers(idx_a, rows_a, sem_a_g)
        wait_gathers(idx_b, rows_b, sem_b_g)
        compute(t + 1, wgt_b, rows_b)
        fire_io(t + 3, idx_b, wgt_b, sem_b_io)
        return carry

    lax.fori_loop(0, _NCHUNK // 2, step, 0)
    wait_gathers(idx_a, rows_a, sem_a_g)
    wait_io(_NCHUNK + 1, idx_b, wgt_b, sem_b_io)


def _sc_gather(idx2, wgt2, val_rows):
    mesh = plsc.VectorSubcoreMesh(core_axis_name="c", subcore_axis_name="s")
    return pl.kernel(
        _sc_body,
        out_type=jax.ShapeDtypeStruct((_QH * _D,), jnp.float32),
        mesh=mesh,
        compiler_params=pltpu.CompilerParams(
            needs_layout_passes=False, use_tc_tiling_on_sc=False
        ),
        scratch_types=[
            pltpu.VMEM((8, 128), jnp.int32),
            pltpu.VMEM((8, 128), jnp.int32),
            pltpu.VMEM((_CH * 64 + 32,), jnp.float32),
            pltpu.VMEM((_CH * 64 + 32,), jnp.float32),
            pltpu.VMEM((_CH * 64, _D), jnp.bfloat16),
            pltpu.VMEM((_CH * 64, _D), jnp.bfloat16),
            pltpu.VMEM((_CH * _D,), jnp.float32),
            pltpu.SemaphoreType.DMA,
            pltpu.SemaphoreType.DMA,
            pltpu.SemaphoreType.DMA,
            pltpu.SemaphoreType.DMA,
        ],
    )(idx2, wgt2, val_rows)


def kernel(query, reference_points, input_flatten, input_spatial_shapes,
           input_level_start_index, W_value, b_value, W_off, b_off, W_attn,
           b_attn, W_out, b_out):
    val_rows = _mm(
        input_flatten.reshape(_N * _HW, _C), W_value[:, _VAL_PERM],
        b_value[_VAL_PERM], out_dtype=jnp.bfloat16,
    ).reshape(_QH, _D)

    rx3 = jnp.repeat(reference_points[..., 0], _P, axis=2).reshape(
        _N * _NQB, _QB, 16
    )
    ry3 = jnp.repeat(reference_points[..., 1], _P, axis=2).reshape(
        _N * _NQB, _QB, 16
    )
    query3 = query.reshape(_N * _NQB, _QB, _C)
    woff = W_off[:, _OFF_PERM]
    boff = b_off[_OFF_PERM].reshape(1, _C)

    idx_out, wgt_out = _prep(
        query3, rx3, ry3, woff, boff, W_attn, b_attn.reshape(1, 128)
    )
    idx2 = idx_out.reshape(_QH * 64 // 128, 128)
    wgt2 = wgt_out.reshape(_QH * 64)

    sampled = _sc_gather(idx2, wgt2, val_rows)

    out = _mm(sampled.reshape(_N * _Q, _C), W_out, b_out)
    return out.reshape(_N, _Q, _C)


# speedup vs baseline: 2.0079x; 1.5695x over previous
import jax
import jax.numpy as jnp
import numpy as np
from jax import lax
from jax.experimental import pallas as pl
from jax.experimental.pallas import tpu as pltpu
from jax.experimental.pallas import tpu_sc as plsc

_N, _Q, _C = 2, 5440, 256
_L, _H, _P = 4, 8, 4
_D = _C // _H
_SPATIAL = ((64, 64), (32, 32), (16, 16), (8, 8))
_HW = sum(h * w for h, w in _SPATIAL)
_QH = _N * _Q * _H
_QB = 320
_NQB = _Q // _QB
_MB = 640

_WV = np.tile(
    np.repeat(np.array([w for (h, w) in _SPATIAL], np.float32), _P), _H
)
_HV = np.tile(
    np.repeat(np.array([h for (h, w) in _SPATIAL], np.float32), _P), _H
)
_STARTV = np.tile(
    np.repeat(
        np.cumsum([0] + [h * w for h, w in _SPATIAL])[:-1].astype(np.float32),
        _P,
    ),
    _H,
)
_FCONST = np.zeros((8, 128), np.float32)
_FCONST[0] = _WV
_FCONST[1] = _HV
_FCONST[2] = 1.0 / _WV
_FCONST[3] = 1.0 / _HV
_FCONST[4] = _STARTV
_FCONST[5] = np.repeat(np.arange(_H, dtype=np.float32), 16)

_OFF_PERM = np.empty(_C, np.int64)
for _h in range(_H):
    for _xy in range(2):
        for _l in range(_L):
            for _p in range(_P):
                _OFF_PERM[_xy * 128 + _h * 16 + _l * 4 + _p] = (
                    ((_h * _L + _l) * _P + _p) * 2 + _xy
                )

_VAL_PERM = np.empty(_C, np.int64)
for _h in range(_H):
    for _t in range(16):
        _VAL_PERM[_h * 32 + 2 * _t] = _h * 32 + _t
        _VAL_PERM[_h * 32 + 2 * _t + 1] = _h * 32 + 16 + _t

_NQ = _N * _Q
_NW = 32
_PW = _NQ // _NW
_CH = 2
_NCHUNK = _PW // _CH


def _mm_body(x_ref, w_ref, b_ref, o_ref):
    o_ref[...] = (
        jnp.dot(x_ref[...], w_ref[...], preferred_element_type=jnp.float32)
        + b_ref[...]
    ).astype(o_ref.dtype)


def _mm(x, w, b, out_dtype=jnp.float32):
    m, k = x.shape
    n = w.shape[1]
    return pl.pallas_call(
        _mm_body,
        grid=(m // _MB,),
        in_specs=[
            pl.BlockSpec((_MB, k), lambda i: (i, 0)),
            pl.BlockSpec((k, n), lambda i: (0, 0)),
            pl.BlockSpec((1, n), lambda i: (0, 0)),
        ],
        out_specs=pl.BlockSpec((_MB, n), lambda i: (i, 0)),
        out_shape=jax.ShapeDtypeStruct((m, n), out_dtype),
    )(x, w, b.reshape(1, n))


def _prep_body(q_ref, rx_ref, ry_ref, woff_ref, boff_ref, wattn_ref, battn_ref,
               fc_ref, idx_ref, wgt_ref):
    g = pl.program_id(0)
    n = g // _NQB
    q = q_ref[0]
    off = (
        jnp.dot(q, woff_ref[...], preferred_element_type=jnp.float32)
        + boff_ref[...]
    )
    att = (
        jnp.dot(q, wattn_ref[...], preferred_element_type=jnp.float32)
        + battn_ref[...]
    )
    rx = jnp.concatenate([rx_ref[0]] * _H, axis=1)
    ry = jnp.concatenate([ry_ref[0]] * _H, axis=1)
    fc = fc_ref[...]
    wv = fc[0:1, :]
    hv = fc[1:2, :]
    winv = fc[2:3, :]
    hinv = fc[3:4, :]
    wvi = wv.astype(jnp.int32)
    hvi = hv.astype(jnp.int32)
    startv = fc[4:5, :].astype(jnp.int32)
    h_lane = fc[5:6, :].astype(jnp.int32)
    e = jnp.exp(att)
    grp = (
        lax.broadcasted_iota(jnp.int32, (128, _H), 0) // 16
        == lax.broadcasted_iota(jnp.int32, (128, _H), 1)
    ).astype(jnp.float32)
    grp_t = (
        lax.broadcasted_iota(jnp.int32, (_H, 128), 0)
        == lax.broadcasted_iota(jnp.int32, (_H, 128), 1) // 16
    ).astype(jnp.float32)
    s8 = jnp.dot(e, grp, preferred_element_type=jnp.float32)
    sm = e * jnp.dot(1.0 / s8, grp_t, preferred_element_type=jnp.float32)
    ox = off[:, 0:128]
    oy = off[:, 128:256]
    x = (rx + ox * winv) * wv - 0.5
    y = (ry + oy * hinv) * hv - 0.5
    x0f = jnp.floor(x)
    y0f = jnp.floor(y)
    lx = x - x0f
    ly = y - y0f
    x0 = x0f.astype(jnp.int32)
    y0 = y0f.astype(jnp.int32)
    idx_parts = []
    w_parts = []
    for dy, dx in ((0, 0), (0, 1), (1, 0), (1, 1)):
        xx = x0 + dx
        yy = y0 + dy
        valid = (xx >= 0) & (xx < wvi) & (yy >= 0) & (yy < hvi)
        xc = jnp.clip(xx, 0, wvi - 1)
        yc = jnp.clip(yy, 0, hvi - 1)
        sp = yc * wvi + xc + startv + n * _HW
        idx_parts.append(sp * _H + h_lane)
        bw = (ly if dy else (1.0 - ly)) * (lx if dx else (1.0 - lx))
        w_parts.append(sm * bw * valid.astype(jnp.float32))
    idx_ref[0] = jnp.concatenate(idx_parts, axis=1)
    wgt_ref[0] = jnp.concatenate(w_parts, axis=1)


def _prep(query3, rx3, ry3, woff, boff, wattn, battn):
    g = _N * _NQB
    return pl.pallas_call(
        _prep_body,
        grid=(g,),
        in_specs=[
            pl.BlockSpec((1, _QB, _C), lambda i: (i, 0, 0)),
            pl.BlockSpec((1, _QB, 16), lambda i: (i, 0, 0)),
            pl.BlockSpec((1, _QB, 16), lambda i: (i, 0, 0)),
            pl.BlockSpec((_C, _C), lambda i: (0, 0)),
            pl.BlockSpec((1, _C), lambda i: (0, 0)),
            pl.BlockSpec((_C, 128), lambda i: (0, 0)),
            pl.BlockSpec((1, 128), lambda i: (0, 0)),
            pl.BlockSpec((8, 128), lambda i: (0, 0)),
        ],
        out_specs=[
            pl.BlockSpec((1, _QB, 512), lambda i: (i, 0, 0)),
            pl.BlockSpec((1, _QB, 512), lambda i: (i, 0, 0)),
        ],
        out_shape=[
            jax.ShapeDtypeStruct((g, _QB, 512), jnp.int32),
            jax.ShapeDtypeStruct((g, _QB, 512), jnp.float32),
        ],
    )(query3, rx3, ry3, woff, boff, wattn, battn, jnp.asarray(_FCONST))


def _sc_body(idx_hbm, wgt_hbm, val_hbm, out_hbm,
             idx_a, idx_b, wgt_a, wgt_b, rows_a, rows_b, out_v,
             sem_a_io, sem_b_io, sem_a_g, sem_b_g):
    cid = lax.axis_index("c")
    sid = lax.axis_index("s")
    wid = sid * 2 + cid
    base0 = wid * _PW
    iota = lax.iota(jnp.int32, 16)
    zero = jnp.zeros((16,), jnp.float32)

    def chunk_off(c):
        c = lax.rem(c, _NCHUNK)
        return pl.multiple_of(base0 + c * _CH, _CH)

    def fire_io(c, idx_v, wgt_v, sem):
        ib = chunk_off(c)
        row128 = pl.multiple_of((ib * 512) // 128, 8)
        cp1 = pltpu.make_async_copy(idx_hbm.at[pl.ds(row128, 8)], idx_v, sem)
        cp2 = pltpu.make_async_copy(
            wgt_hbm.at[pl.ds(pl.multiple_of(ib * 512, 1024), _CH * 512)],
            wgt_v.at[pl.ds(0, _CH * 512)], sem,
        )
        cp1.start()
        cp2.start()

    def wait_io(c, idx_v, wgt_v, sem):
        ib = chunk_off(c)
        row128 = pl.multiple_of((ib * 512) // 128, 8)
        pltpu.make_async_copy(idx_hbm.at[pl.ds(row128, 8)], idx_v, sem).wait()
        pltpu.make_async_copy(
            wgt_hbm.at[pl.ds(pl.multiple_of(ib * 512, 1024), _CH * 512)],
            wgt_v.at[pl.ds(0, _CH * 512)], sem,
        ).wait()

    def fire_gathers(idx_v, rows_v, sem):
        for j in range(8):
            pltpu.make_async_copy(
                val_hbm.at[idx_v.at[j]], rows_v.at[pl.ds(j * 128, 128)], sem
            ).start()

    def wait_gathers(idx_v, rows_v, sem):
        for j in range(8):
            pltpu.make_async_copy(
                val_hbm.at[idx_v.at[j]], rows_v.at[pl.ds(j * 128, 128)], sem
            ).wait()

    def compute(c, wgt_v, rows_v):
        ib = chunk_off(c)
        for k2 in (0, 1):
            for hg in (0, 4):

                def jbody(k, accs, k2=k2, hg=hg):
                    base = k2 * 512 + lax.shift_left(
                        lax.bitwise_and(k, 48), 3
                    ) + lax.bitwise_and(k, 15)
                    new = []
                    for t in range(4):
                        fl = base + (hg + t) * 16
                        w0 = wgt_v[pl.ds(fl, 16)][0]
                        l0, l1 = plsc.unpack(
                            rows_v[fl, 0:32],
                            format=plsc.PackFormat.INTERLEAVED,
                        )
                        new += [accs[2 * t] + w0 * l0, accs[2 * t + 1] + w0 * l1]
                    return tuple(new)

                accs = lax.fori_loop(0, 64, jbody, (zero,) * 8, unroll=2)
                for t in range(4):
                    o0 = k2 * 256 + (hg + t) * 32
                    out_v[pl.ds(o0, 16)] = accs[2 * t]
                    out_v[pl.ds(o0 + 16, 16)] = accs[2 * t + 1]
        pltpu.sync_copy(
            out_v, out_hbm.at[pl.ds(pl.multiple_of(ib * 256, 512), _CH * 256)]
        )

    fire_io(0, idx_a, wgt_a, sem_a_io)
    fire_io(1, idx_b, wgt_b, sem_b_io)
    wait_io(0, idx_a, wgt_a, sem_a_io)
    fire_gathers(idx_a, rows_a, sem_a_g)

    def step(k, carry):
        t = k * 2
        wait_io(t + 1, idx_b, wgt_b, sem_b_io)
        fire_gathers(idx_b, rows_b, sem_b_g)
        wait_gathers(idx_a, rows_a, sem_a_g)
        compute(t, wgt_a, rows_a)
        fire_io(t + 2, idx_a, wgt_a, sem_a_io)

        wait_io(t + 2, idx_a, wgt_a, sem_a_io)
        fire_gathers(idx_a, rows_a, sem_a_g)
        wait_gathers(idx_b, rows_b, sem_b_g)
        compute(t + 1, wgt_b, rows_b)
        fire_io(t + 3, idx_b, wgt_b, sem_b_io)
        return carry

    lax.fori_loop(0, _NCHUNK // 2, step, 0)
    wait_gathers(idx_a, rows_a, sem_a_g)
    wait_io(_NCHUNK + 1, idx_b, wgt_b, sem_b_io)


def _sc_gather(idx2, wgt2, val_rows):
    mesh = plsc.VectorSubcoreMesh(core_axis_name="c", subcore_axis_name="s")
    return pl.kernel(
        _sc_body,
        out_type=jax.ShapeDtypeStruct((_NQ * _C,), jnp.float32),
        mesh=mesh,
        compiler_params=pltpu.CompilerParams(
            needs_layout_passes=False, use_tc_tiling_on_sc=False
        ),
        scratch_types=[
            pltpu.VMEM((8, 128), jnp.int32),
            pltpu.VMEM((8, 128), jnp.int32),
            pltpu.VMEM((_CH * 512 + 32,), jnp.float32),
            pltpu.VMEM((_CH * 512 + 32,), jnp.float32),
            pltpu.VMEM((_CH * 512, _D), jnp.bfloat16),
            pltpu.VMEM((_CH * 512, _D), jnp.bfloat16),
            pltpu.VMEM((_CH * _C,), jnp.float32),
            pltpu.SemaphoreType.DMA,
            pltpu.SemaphoreType.DMA,
            pltpu.SemaphoreType.DMA,
            pltpu.SemaphoreType.DMA,
        ],
    )(idx2, wgt2, val_rows)


def kernel(query, reference_points, input_flatten, input_spatial_shapes,
           input_level_start_index, W_value, b_value, W_off, b_off, W_attn,
           b_attn, W_out, b_out):
    val_rows = _mm(
        input_flatten.reshape(_N * _HW, _C), W_value[:, _VAL_PERM],
        b_value[_VAL_PERM], out_dtype=jnp.bfloat16,
    ).reshape(_QH, _D)

    rx3 = jnp.repeat(reference_points[..., 0], _P, axis=2).reshape(
        _N * _NQB, _QB, 16
    )
    ry3 = jnp.repeat(reference_points[..., 1], _P, axis=2).reshape(
        _N * _NQB, _QB, 16
    )
    query3 = query.reshape(_N * _NQB, _QB, _C)
    woff = W_off[:, _OFF_PERM]
    boff = b_off[_OFF_PERM].reshape(1, _C)

    idx_out, wgt_out = _prep(
        query3, rx3, ry3, woff, boff, W_attn, b_attn.reshape(1, 128)
    )
    idx2 = idx_out.reshape(_NQ * 512 // 128, 128)
    wgt2 = wgt_out.reshape(_NQ * 512)

    sampled = _sc_gather(idx2, wgt2, val_rows)

    out = _mm(sampled.reshape(_N * _Q, _C), W_out, b_out)
    return out.reshape(_N, _Q, _C)


# speedup vs baseline: 2.0910x; 1.0414x over previous
import jax
import jax.numpy as jnp
import numpy as np
from jax import lax
from jax.experimental import pallas as pl
from jax.experimental.pallas import tpu as pltpu
from jax.experimental.pallas import tpu_sc as plsc

_N, _Q, _C = 2, 5440, 256
_L, _H, _P = 4, 8, 4
_D = _C // _H
_SPATIAL = ((64, 64), (32, 32), (16, 16), (8, 8))
_HW = sum(h * w for h, w in _SPATIAL)
_QH = _N * _Q * _H
_QB = 320
_NQB = _Q // _QB
_MB = 640

_WV = np.tile(
    np.repeat(np.array([w for (h, w) in _SPATIAL], np.float32), _P), _H
)
_HV = np.tile(
    np.repeat(np.array([h for (h, w) in _SPATIAL], np.float32), _P), _H
)
_STARTV = np.tile(
    np.repeat(
        np.cumsum([0] + [h * w for h, w in _SPATIAL])[:-1].astype(np.float32),
        _P,
    ),
    _H,
)
_FCONST = np.zeros((8, 128), np.float32)
_FCONST[0] = _WV
_FCONST[1] = _HV
_FCONST[2] = 1.0 / _WV
_FCONST[3] = 1.0 / _HV
_FCONST[4] = _STARTV
_FCONST[5] = np.repeat(np.arange(_H, dtype=np.float32), 16)

_OFF_PERM = np.empty(_C, np.int64)
for _h in range(_H):
    for _xy in range(2):
        for _l in range(_L):
            for _p in range(_P):
                _OFF_PERM[_xy * 128 + _h * 16 + _l * 4 + _p] = (
                    ((_h * _L + _l) * _P + _p) * 2 + _xy
                )

_VAL_PERM = np.empty(_C, np.int64)
for _h in range(_H):
    for _t in range(16):
        _VAL_PERM[_h * 32 + 2 * _t] = _h * 32 + _t
        _VAL_PERM[_h * 32 + 2 * _t + 1] = _h * 32 + 16 + _t

_NQ = _N * _Q
_NW = 32
_PW = _NQ // _NW
_CH = 4
_NCHUNK = _PW // _CH
_NSTREAM = _CH * 512 // 128


def _mm_body(x_ref, w_ref, b_ref, o_ref):
    o_ref[...] = (
        jnp.dot(x_ref[...], w_ref[...], preferred_element_type=jnp.float32)
        + b_ref[...]
    ).astype(o_ref.dtype)


def _mm(x, w, b, out_dtype=jnp.float32):
    m, k = x.shape
    n = w.shape[1]
    return pl.pallas_call(
        _mm_body,
        grid=(m // _MB,),
        in_specs=[
            pl.BlockSpec((_MB, k), lambda i: (i, 0)),
            pl.BlockSpec((k, n), lambda i: (0, 0)),
            pl.BlockSpec((1, n), lambda i: (0, 0)),
        ],
        out_specs=pl.BlockSpec((_MB, n), lambda i: (i, 0)),
        out_shape=jax.ShapeDtypeStruct((m, n), out_dtype),
    )(x, w, b.reshape(1, n))


def _prep_body(q_ref, rx_ref, ry_ref, woff_ref, boff_ref, wattn_ref, battn_ref,
               fc_ref, idx_ref, wgt_ref):
    g = pl.program_id(0)
    n = g // _NQB
    q = q_ref[0]
    off = (
        jnp.dot(q, woff_ref[...], preferred_element_type=jnp.float32)
        + boff_ref[...]
    )
    att = (
        jnp.dot(q, wattn_ref[...], preferred_element_type=jnp.float32)
        + battn_ref[...]
    )
    rx = jnp.concatenate([rx_ref[0]] * _H, axis=1)
    ry = jnp.concatenate([ry_ref[0]] * _H, axis=1)
    fc = fc_ref[...]
    wv = fc[0:1, :]
    hv = fc[1:2, :]
    winv = fc[2:3, :]
    hinv = fc[3:4, :]
    wvi = wv.astype(jnp.int32)
    hvi = hv.astype(jnp.int32)
    startv = fc[4:5, :].astype(jnp.int32)
    h_lane = fc[5:6, :].astype(jnp.int32)
    e = jnp.exp(att)
    grp = (
        lax.broadcasted_iota(jnp.int32, (128, _H), 0) // 16
        == lax.broadcasted_iota(jnp.int32, (128, _H), 1)
    ).astype(jnp.float32)
    grp_t = (
        lax.broadcasted_iota(jnp.int32, (_H, 128), 0)
        == lax.broadcasted_iota(jnp.int32, (_H, 128), 1) // 16
    ).astype(jnp.float32)
    s8 = jnp.dot(e, grp, preferred_element_type=jnp.float32)
    sm = e * jnp.dot(1.0 / s8, grp_t, preferred_element_type=jnp.float32)
    ox = off[:, 0:128]
    oy = off[:, 128:256]
    x = (rx + ox * winv) * wv - 0.5
    y = (ry + oy * hinv) * hv - 0.5
    x0f = jnp.floor(x)
    y0f = jnp.floor(y)
    lx = x - x0f
    ly = y - y0f
    x0 = x0f.astype(jnp.int32)
    y0 = y0f.astype(jnp.int32)
    idx_parts = []
    w_parts = []
    for dy, dx in ((0, 0), (0, 1), (1, 0), (1, 1)):
        xx = x0 + dx
        yy = y0 + dy
        valid = (xx >= 0) & (xx < wvi) & (yy >= 0) & (yy < hvi)
        xc = jnp.clip(xx, 0, wvi - 1)
        yc = jnp.clip(yy, 0, hvi - 1)
        sp = yc * wvi + xc + startv + n * _HW
        idx_parts.append(sp * _H + h_lane)
        bw = (ly if dy else (1.0 - ly)) * (lx if dx else (1.0 - lx))
        w_parts.append(sm * bw * valid.astype(jnp.float32))
    idx_ref[0] = jnp.concatenate(idx_parts, axis=1)
    wgt_ref[0] = jnp.concatenate(w_parts, axis=1)


def _prep(query3, rx3, ry3, woff, boff, wattn, battn):
    g = _N * _NQB
    return pl.pallas_call(
        _prep_body,
        grid=(g,),
        in_specs=[
            pl.BlockSpec((1, _QB, _C), lambda i: (i, 0, 0)),
            pl.BlockSpec((1, _QB, 16), lambda i: (i, 0, 0)),
            pl.BlockSpec((1, _QB, 16), lambda i: (i, 0, 0)),
            pl.BlockSpec((_C, _C), lambda i: (0, 0)),
            pl.BlockSpec((1, _C), lambda i: (0, 0)),
            pl.BlockSpec((_C, 128), lambda i: (0, 0)),
            pl.BlockSpec((1, 128), lambda i: (0, 0)),
            pl.BlockSpec((8, 128), lambda i: (0, 0)),
        ],
        out_specs=[
            pl.BlockSpec((1, _QB, 512), lambda i: (i, 0, 0)),
            pl.BlockSpec((1, _QB, 512), lambda i: (i, 0, 0)),
        ],
        out_shape=[
            jax.ShapeDtypeStruct((g, _QB, 512), jnp.int32),
            jax.ShapeDtypeStruct((g, _QB, 512), jnp.float32),
        ],
    )(query3, rx3, ry3, woff, boff, wattn, battn, jnp.asarray(_FCONST))


def _sc_body(idx_hbm, wgt_hbm, val_hbm, out_hbm,
             idx_a, idx_b, wgt_a, wgt_b, rows_a, rows_b, out_v,
             sem_a_io, sem_b_io, sem_a_g, sem_b_g):
    cid = lax.axis_index("c")
    sid = lax.axis_index("s")
    wid = sid * 2 + cid
    base0 = wid * _PW
    iota = lax.iota(jnp.int32, 16)
    zero = jnp.zeros((16,), jnp.float32)

    def chunk_off(c):
        c = lax.rem(c, _NCHUNK)
        return pl.multiple_of(base0 + c * _CH, _CH)

    def fire_io(c, idx_v, wgt_v, sem):
        ib = chunk_off(c)
        row128 = pl.multiple_of((ib * 512) // 128, _NSTREAM)
        cp1 = pltpu.make_async_copy(
            idx_hbm.at[pl.ds(row128, _NSTREAM)], idx_v, sem
        )
        cp2 = pltpu.make_async_copy(
            wgt_hbm.at[pl.ds(pl.multiple_of(ib * 512, 1024), _CH * 512)],
            wgt_v.at[pl.ds(0, _CH * 512)], sem,
        )
        cp1.start()
        cp2.start()

    def wait_io(c, idx_v, wgt_v, sem):
        ib = chunk_off(c)
        row128 = pl.multiple_of((ib * 512) // 128, _NSTREAM)
        pltpu.make_async_copy(
            idx_hbm.at[pl.ds(row128, _NSTREAM)], idx_v, sem
        ).wait()
        pltpu.make_async_copy(
            wgt_hbm.at[pl.ds(pl.multiple_of(ib * 512, 1024), _CH * 512)],
            wgt_v.at[pl.ds(0, _CH * 512)], sem,
        ).wait()

    def fire_gathers(idx_v, rows_v, sem):
        for j in range(_NSTREAM):
            pltpu.make_async_copy(
                val_hbm.at[idx_v.at[j]], rows_v.at[pl.ds(j * 128, 128)], sem
            ).start()

    def wait_gathers(idx_v, rows_v, sem):
        for j in range(_NSTREAM):
            pltpu.make_async_copy(
                val_hbm.at[idx_v.at[j]], rows_v.at[pl.ds(j * 128, 128)], sem
            ).wait()

    def compute(c, wgt_v, rows_v):
        ib = chunk_off(c)
        for k2 in range(_CH):
            for hg in (0, 4):

                def jbody(k, accs, k2=k2, hg=hg):
                    base = k2 * 512 + lax.shift_left(
                        lax.bitwise_and(k, 48), 3
                    ) + lax.bitwise_and(k, 15)
                    new = []
                    for t in range(4):
                        fl = base + (hg + t) * 16
                        w0 = wgt_v[pl.ds(fl, 16)][0]
                        l0, l1 = plsc.unpack(
                            rows_v[fl, 0:32],
                            format=plsc.PackFormat.INTERLEAVED,
                        )
                        new += [accs[2 * t] + w0 * l0, accs[2 * t + 1] + w0 * l1]
                    return tuple(new)

                accs = lax.fori_loop(0, 64, jbody, (zero,) * 8, unroll=2)
                for t in range(4):
                    o0 = k2 * 256 + (hg + t) * 32
                    out_v[pl.ds(o0, 16)] = accs[2 * t]
                    out_v[pl.ds(o0 + 16, 16)] = accs[2 * t + 1]
        pltpu.sync_copy(
            out_v, out_hbm.at[pl.ds(pl.multiple_of(ib * 256, 512), _CH * 256)]
        )

    fire_io(0, idx_a, wgt_a, sem_a_io)
    fire_io(1, idx_b, wgt_b, sem_b_io)
    wait_io(0, idx_a, wgt_a, sem_a_io)
    fire_gathers(idx_a, rows_a, sem_a_g)

    def step(k, carry):
        t = k * 2
        wait_io(t + 1, idx_b, wgt_b, sem_b_io)
        fire_gathers(idx_b, rows_b, sem_b_g)
        wait_gathers(idx_a, rows_a, sem_a_g)
        compute(t, wgt_a, rows_a)
        fire_io(t + 2, idx_a, wgt_a, sem_a_io)

        wait_io(t + 2, idx_a, wgt_a, sem_a_io)
        fire_gathers(idx_a, rows_a, sem_a_g)
        wait_gathers(idx_b, rows_b, sem_b_g)
        compute(t + 1, wgt_b, rows_b)
        fire_io(t + 3, idx_b, wgt_b, sem_b_io)
        return carry

    lax.fori_loop(0, _NCHUNK // 2, step, 0)
    wait_gathers(idx_a, rows_a, sem_a_g)
    wait_io(_NCHUNK + 1, idx_b, wgt_b, sem_b_io)


def _sc_gather(idx2, wgt2, val_rows):
    mesh = plsc.VectorSubcoreMesh(core_axis_name="c", subcore_axis_name="s")
    return pl.kernel(
        _sc_body,
        out_type=jax.ShapeDtypeStruct((_NQ * _C,), jnp.float32),
        mesh=mesh,
        compiler_params=pltpu.CompilerParams(
            needs_layout_passes=False, use_tc_tiling_on_sc=False
        ),
        scratch_types=[
            pltpu.VMEM((_NSTREAM, 128), jnp.int32),
            pltpu.VMEM((_NSTREAM, 128), jnp.int32),
            pltpu.VMEM((_CH * 512 + 32,), jnp.float32),
            pltpu.VMEM((_CH * 512 + 32,), jnp.float32),
            pltpu.VMEM((_CH * 512, _D), jnp.bfloat16),
            pltpu.VMEM((_CH * 512, _D), jnp.bfloat16),
            pltpu.VMEM((_CH * _C,), jnp.float32),
            pltpu.SemaphoreType.DMA,
            pltpu.SemaphoreType.DMA,
            pltpu.SemaphoreType.DMA,
            pltpu.SemaphoreType.DMA,
        ],
    )(idx2, wgt2, val_rows)


def kernel(query, reference_points, input_flatten, input_spatial_shapes,
           input_level_start_index, W_value, b_value, W_off, b_off, W_attn,
           b_attn, W_out, b_out):
    val_rows = _mm(
        input_flatten.reshape(_N * _HW, _C), W_value[:, _VAL_PERM],
        b_value[_VAL_PERM], out_dtype=jnp.bfloat16,
    ).reshape(_QH, _D)

    rx3 = jnp.repeat(reference_points[..., 0], _P, axis=2).reshape(
        _N * _NQB, _QB, 16
    )
    ry3 = jnp.repeat(reference_points[..., 1], _P, axis=2).reshape(
        _N * _NQB, _QB, 16
    )
    query3 = query.reshape(_N * _NQB, _QB, _C)
    woff = W_off[:, _OFF_PERM]
    boff = b_off[_OFF_PERM].reshape(1, _C)

    idx_out, wgt_out = _prep(
        query3, rx3, ry3, woff, boff, W_attn, b_attn.reshape(1, 128)
    )
    idx2 = idx_out.reshape(_NQ * 512 // 128, 128)
    wgt2 = wgt_out.reshape(_NQ * 512)

    sampled = _sc_gather(idx2, wgt2, val_rows)

    out = _mm(sampled.reshape(_N * _Q, _C), W_out, b_out)
    return out.reshape(_N, _Q, _C)
